# baseline (device time: 21360 ns/iter reference)
import jax
import jax.numpy as jnp
from jax import lax
from jax.experimental import pallas as pl
from jax.experimental.pallas import tpu as pltpu

N_DEV = 8
N_TOK = 1024
D_IN = 256
D_OUT = 512
E_LOCAL = 4
N_EXP = 32
CHUNK = N_TOK // N_DEV


def kernel(x, router_W, route_idx, expert_W):
    def body(x_ref, rw_ref, idx_ref, ew_ref, out_ref,
             coeff_ref, send_ref, recv_ref, send_sems, recv_sems):
        my = lax.axis_index("i")

        barrier_sem = pltpu.get_barrier_semaphore()
        for d in range(1, N_DEV):
            pl.semaphore_signal(
                barrier_sem, inc=1,
                device_id=(lax.rem(my + d, N_DEV),),
                device_id_type=pl.DeviceIdType.MESH,
            )
        pl.semaphore_wait(barrier_sem, N_DEV - 1)

        xf = x_ref[:, :]
        scores = jnp.dot(
            xf.astype(jnp.bfloat16), rw_ref[:, :].astype(jnp.bfloat16),
            preferred_element_type=jnp.float32,
        )
        m = jnp.max(scores, axis=-1, keepdims=True)
        p = jnp.exp(scores - m)
        eids = lax.broadcasted_iota(jnp.int32, (N_TOK, N_EXP), 1)
        i0 = idx_ref[:, 0:1]
        i1 = idx_ref[:, 1:2]
        p0 = jnp.sum(jnp.where(eids == i0, p, 0.0), axis=1, keepdims=True)
        p1 = jnp.sum(jnp.where(eids == i1, p, 0.0), axis=1, keepdims=True)
        gs = p0 + p1
        w0 = p0 / gs
        w1 = p1 / gs

        cks = []
        for k in range(E_LOCAL):
            e = my * E_LOCAL + k
            cks.append(w0 * (i0 == e).astype(jnp.float32)
                       + w1 * (i1 == e).astype(jnp.float32))
        coeff_ref[:, :] = jnp.concatenate(cks, axis=1)

        def chunk_partial(t):
            xc = x_ref[pl.ds(t * CHUNK, CHUNK), :]
            cc = coeff_ref[pl.ds(t * CHUNK, CHUNK), :]
            pc = jnp.zeros((CHUNK, D_OUT), jnp.float32)
            for k in range(E_LOCAL):
                xk = (xc * cc[:, k:k + 1]).astype(jnp.bfloat16)
                pc = pc + jnp.dot(
                    xk, ew_ref[k, :, :].astype(jnp.bfloat16),
                    preferred_element_type=jnp.float32,
                )
            return pc

        rdmas = []
        for d in range(1, N_DEV):
            t = lax.rem(my + d, N_DEV)
            send_ref[d, :, :] = chunk_partial(t).astype(jnp.bfloat16)
            rdma = pltpu.make_async_remote_copy(
                src_ref=send_ref.at[d],
                dst_ref=recv_ref.at[d],
                send_sem=send_sems.at[d],
                recv_sem=recv_sems.at[d],
                device_id=(t,),
                device_id_type=pl.DeviceIdType.MESH,
            )
            rdma.start()
            rdmas.append(rdma)

        acc = chunk_partial(my)
        for d in range(1, N_DEV):
            rdmas[d - 1].wait_recv()
            acc = acc + recv_ref[d, :, :].astype(jnp.float32)
        out_ref[:, :] = acc
        for r in rdmas:
            r.wait_send()

    return pl.pallas_call(
        body,
        out_shape=jax.ShapeDtypeStruct((CHUNK, D_OUT), jnp.float32),
        in_specs=[pl.BlockSpec(memory_space=pltpu.VMEM)] * 4,
        out_specs=pl.BlockSpec(memory_space=pltpu.VMEM),
        scratch_shapes=[
            pltpu.VMEM((N_TOK, E_LOCAL), jnp.float32),
            pltpu.VMEM((N_DEV, CHUNK, D_OUT), jnp.bfloat16),
            pltpu.VMEM((N_DEV, CHUNK, D_OUT), jnp.bfloat16),
            pltpu.SemaphoreType.DMA((N_DEV,)),
            pltpu.SemaphoreType.DMA((N_DEV,)),
        ],
        compiler_params=pltpu.CompilerParams(collective_id=0),
    )(x, router_W, route_idx, expert_W)
